# baseline (speedup 1.0000x reference)
import functools

import jax
import jax.numpy as jnp
from jax import lax
from jax.experimental import pallas as pl
from jax.experimental.pallas import tpu as pltpu
from jax.experimental.pallas import tpu_sc as plsc

NC, NS, L = 2, 16, 16
NW = NC * NS
C = 80


@functools.lru_cache(maxsize=None)
def _build(B, V, D):
    BPW = B // NW
    NCH = BPW // C
    assert BPW * NW == B and NCH * C == BPW and NCH % 2 == 0
    assert C % L == 0 and D % L == 0

    mesh = plsc.VectorSubcoreMesh(core_axis_name="c", subcore_axis_name="s")

    @functools.partial(
        pl.kernel,
        out_type=jax.ShapeDtypeStruct((B * D,), jnp.float32),
        mesh=mesh,
        compiler_params=pltpu.CompilerParams(needs_layout_passes=False),
        scratch_types=[
            pltpu.VMEM((BPW,), jnp.int32),
            pltpu.VMEM((V * D,), jnp.float32),
            pltpu.VMEM((C * D,), jnp.float32),
            pltpu.VMEM((C * D,), jnp.float32),
            pltpu.SemaphoreType.DMA,
            pltpu.SemaphoreType.DMA,
        ],
    )
    def emb(idx_hbm, table_hbm, out_hbm, idx_v, table_v, rows0, rows1, o0, o1):
        rows = (rows0, rows1)
        osem = (o0, o1)
        wid = lax.axis_index("s") * NC + lax.axis_index("c")
        base = wid * BPW
        pltpu.sync_copy(idx_hbm.at[pl.ds(base, BPW)], idx_v)
        pltpu.sync_copy(table_hbm, table_v)

        def wait_o(b):
            pltpu.make_async_copy(rows[b], out_hbm.at[pl.ds(0, C * D)],
                                  osem[b]).wait()

        def step(c, b):
            @pl.when(c >= 2)
            def _():
                wait_o(b)

            def gbody(g, carry, b=b):
                idx16 = idx_v[pl.ds(c * C + g * L, L)]
                rb16 = idx16 * D
                rowpos = (g * L) * D
                tbs = [rb16[u] for u in range(L)]
                dsts = [rowpos + u * D for u in range(L)]

                @plsc.parallel_loop(0, D // L, unroll=D // L)
                def jbody(j, tbs=tbs, dsts=dsts, b=b):
                    off = j * L
                    for u in range(L):
                        rows[b][pl.ds(dsts[u] + off, L)] = (
                            table_v[pl.ds(tbs[u] + off, L)])
                return carry

            lax.fori_loop(0, C // L, gbody, 0)

            pltpu.async_copy(rows[b],
                             out_hbm.at[pl.ds((base + c * C) * D, C * D)],
                             osem[b])

        def body(i, carry):
            step(2 * i, 0)
            step(2 * i + 1, 1)
            return carry

        lax.fori_loop(0, NCH // 2, body, 0)
        wait_o(0)
        wait_o(1)

    return emb


def kernel(word_seq, embedding_table):
    s, t = word_seq.shape
    b = s * t
    v, d = embedding_table.shape
    idx = word_seq.reshape(b).astype(jnp.int32)
    table = embedding_table.astype(jnp.float32).reshape(v * d)
    out = _build(b, v, d)(idx, table)
    return out.reshape(s, t, d)


# speedup vs baseline: 1.3304x; 1.3304x over previous
import functools

import jax
import jax.numpy as jnp
from jax import lax
from jax.experimental import pallas as pl
from jax.experimental.pallas import tpu as pltpu
from jax.experimental.pallas import tpu_sc as plsc

NC, NS, L = 2, 16, 16
NW = NC * NS
C = 80


@functools.lru_cache(maxsize=None)
def _build(B, V, D):
    BPW = B // NW
    NCH = BPW // C
    assert BPW * NW == B and NCH * C == BPW and NCH % 2 == 0

    mesh = plsc.VectorSubcoreMesh(core_axis_name="c", subcore_axis_name="s")

    @functools.partial(
        pl.kernel,
        out_type=jax.ShapeDtypeStruct((B, D), jnp.float32),
        mesh=mesh,
        compiler_params=pltpu.CompilerParams(needs_layout_passes=False),
        scratch_types=[
            pltpu.VMEM((BPW,), jnp.int32),
            pltpu.VMEM((C, D), jnp.float32),
            pltpu.VMEM((C, D), jnp.float32),
            pltpu.SemaphoreType.DMA,
            pltpu.SemaphoreType.DMA,
            pltpu.SemaphoreType.DMA,
            pltpu.SemaphoreType.DMA,
        ],
    )
    def emb(idx_hbm, table_hbm, out_hbm, idx_v, rows0, rows1, g0, g1, o0, o1):
        rows = (rows0, rows1)
        gsem = (g0, g1)
        osem = (o0, o1)
        wid = lax.axis_index("s") * NC + lax.axis_index("c")
        base = wid * BPW
        pltpu.sync_copy(idx_hbm.at[pl.ds(base, BPW)], idx_v)

        bias = wid * V

        @plsc.parallel_loop(0, BPW // L, unroll=8)
        def bbody(k):
            idx_v[pl.ds(k * L, L)] = idx_v[pl.ds(k * L, L)] + bias

        def start_g(c, b):
            pltpu.async_copy(table_hbm.at[idx_v.at[pl.ds(c * C, C)]],
                             rows[b], gsem[b])

        def wait_g(b):
            pltpu.make_async_copy(table_hbm.at[idx_v.at[pl.ds(0, C)]],
                                  rows[b], gsem[b]).wait()

        def start_o(c, b):
            pltpu.async_copy(rows[b], out_hbm.at[pl.ds(base + c * C, C)],
                             osem[b])

        def wait_o(b):
            pltpu.make_async_copy(rows[b], out_hbm.at[pl.ds(0, C)],
                                  osem[b]).wait()

        start_g(0, 0)

        def step(c, b):
            wait_g(b)
            start_o(c, b)

            @pl.when(c + 1 < NCH)
            def _():
                @pl.when(c >= 1)
                def _():
                    wait_o(1 - b)
                start_g(c + 1, 1 - b)

        def body(i, carry):
            step(2 * i, 0)
            step(2 * i + 1, 1)
            return carry

        lax.fori_loop(0, NCH // 2, body, 0)
        wait_o(0)
        wait_o(1)

    return emb


def kernel(word_seq, embedding_table):
    s, t = word_seq.shape
    b = s * t
    v, d = embedding_table.shape
    idx = word_seq.reshape(b).astype(jnp.int32)
    table = jnp.tile(embedding_table.astype(jnp.float32), (NW, 1))
    out = _build(b, v, d)(idx, table)
    return out.reshape(s, t, d)
